# baseline (device time: 54665 ns/iter reference)
import jax
import jax.numpy as jnp
from jax import lax
from jax.experimental import pallas as pl
from jax.experimental.pallas import tpu as pltpu

TILE = 256
WK = 768
WR = 512


def kernel(x, dest):
    m, n = x.shape
    nt = m // TILE

    my_y = lax.axis_index("y")

    is_send = (dest != my_y).astype(jnp.int32)
    cs = jnp.cumsum(is_send)
    n_send = cs[-1]
    nk = m - n_send
    idx = jnp.arange(m, dtype=jnp.int32)
    r_send = cs - is_send
    r_keep = idx - cs + is_send

    koff = jnp.where(my_y == 0, 0, n_send)
    okp = jnp.where(is_send == 0, koff + r_keep, -1).astype(jnp.int32)
    osr = jnp.where(is_send == 1, r_send, -1).astype(jnp.int32)

    def rank_windows(rank_vec, marked, target_vec, base):
        starts, wins = [], []
        total = jnp.sum(marked)
        for t in range(nt):
            rlo = jnp.clip(t * TILE - base, 0, jnp.maximum(total - 1, 0))
            match = (rank_vec == rlo) & (marked == 1)
            pos = jnp.sum(jnp.where(match, idx, 0))
            w_t = jnp.clip((pos // 128) * 128, 0, m - WK).astype(jnp.int32)
            starts.append(w_t)
            wins.append(lax.dynamic_slice(target_vec, (w_t,), (WK,)))
        return jnp.stack(starts), jnp.stack(wins)

    is_keep = 1 - is_send
    wk, okw = rank_windows(r_keep, is_keep, okp, koff)
    ws, osw = rank_windows(r_send, is_send, osr, 0)

    def body(ns_ref, wk_ref, ws_ref, xb_ref, okw_ref, osw_ref, out_ref,
             xs_ref, sb_ref, send_sems, recv_sems):
        my_xx = lax.axis_index("x")
        my_yy = lax.axis_index("y")
        my_zz = lax.axis_index("z")
        peer = (my_xx, 1 - my_yy, my_zz)

        barrier = pltpu.get_barrier_semaphore()
        pl.semaphore_signal(
            barrier, inc=1, device_id=peer, device_id_type=pl.DeviceIdType.MESH
        )
        pl.semaphore_wait(barrier, 1)

        ns = ns_ref[0]
        nkk = m - ns
        koff_s = jnp.where(my_yy == 0, 0, ns)
        ro = jnp.where(my_yy == 0, nkk, 0)

        def tile_rdma(t):
            return pltpu.make_async_remote_copy(
                src_ref=xs_ref.at[pl.ds(t * TILE, TILE)],
                dst_ref=sb_ref.at[pl.ds(t * TILE, TILE)],
                send_sem=send_sems.at[t],
                recv_sem=recv_sems.at[t],
                device_id=peer,
                device_id_type=pl.DeviceIdType.MESH,
            )

        def onehot_mm(win_ref, t, w_t, t0):
            d_iota = lax.broadcasted_iota(jnp.int32, (TILE, WK), 0) + t0
            p = (win_ref[t : t + 1, :] == d_iota).astype(jnp.bfloat16)
            return jax.lax.dot_general(
                p,
                xb_ref[pl.ds(w_t, WK), :],
                dimension_numbers=(((1,), (0,)), ((), ())),
                preferred_element_type=jnp.float32,
            )

        for t in range(nt):
            @pl.when(t * TILE < ns)
            def _():
                w_t = pl.multiple_of(ws_ref[t], 128)
                xs_ref[pl.ds(t * TILE, TILE), :] = onehot_mm(
                    osw_ref, t, w_t, t * TILE
                ).astype(jnp.bfloat16)
                tile_rdma(t).start()

        for t in range(nt):
            t0 = t * TILE
            has_keep = (t0 < koff_s + nkk) & (t0 + TILE > koff_s)

            @pl.when(has_keep)
            def _():
                w_t = pl.multiple_of(wk_ref[t], 128)
                out_ref[pl.ds(t0, TILE), :] = onehot_mm(okw_ref, t, w_t, t0)

        for t in range(nt):
            @pl.when(t * TILE < ns)
            def _():
                tile_rdma(t).wait_recv()

        for t in range(nt):
            t0 = t * TILE
            has_keep = (t0 < koff_s + nkk) & (t0 + TILE > koff_s)
            has_recv = (t0 < ro + ns) & (t0 + TILE > ro)
            wr_t = pl.multiple_of(
                jnp.clip((jnp.maximum(t0 - ro, 0) // 128) * 128, 0, m - WR),
                128,
            )
            d_iota = lax.broadcasted_iota(jnp.int32, (TILE, WR), 0) + t0
            j_iota = lax.broadcasted_iota(jnp.int32, (TILE, WR), 1) + wr_t
            band = (j_iota == d_iota - ro) & (j_iota < ns)

            def recv_mm():
                jrow = lax.broadcasted_iota(jnp.int32, (WR, 1), 0) + wr_t
                sbw = jnp.where(
                    jrow < ns, sb_ref[pl.ds(wr_t, WR), :], jnp.bfloat16(0)
                )
                return jax.lax.dot_general(
                    band.astype(jnp.bfloat16),
                    sbw,
                    dimension_numbers=(((1,), (0,)), ((), ())),
                    preferred_element_type=jnp.float32,
                )

            @pl.when(has_recv & jnp.logical_not(has_keep))
            def _():
                out_ref[pl.ds(t0, TILE), :] = recv_mm()

            @pl.when(has_recv & has_keep)
            def _():
                out_ref[pl.ds(t0, TILE), :] = (
                    out_ref[pl.ds(t0, TILE), :] + recv_mm()
                )

        for t in range(nt):
            @pl.when(t * TILE < ns)
            def _():
                tile_rdma(t).wait_send()

    return pl.pallas_call(
        body,
        out_shape=jax.ShapeDtypeStruct((m, n), x.dtype),
        in_specs=[
            pl.BlockSpec(memory_space=pltpu.SMEM),
            pl.BlockSpec(memory_space=pltpu.SMEM),
            pl.BlockSpec(memory_space=pltpu.SMEM),
            pl.BlockSpec(memory_space=pltpu.VMEM),
            pl.BlockSpec(memory_space=pltpu.VMEM),
            pl.BlockSpec(memory_space=pltpu.VMEM),
        ],
        out_specs=pl.BlockSpec(memory_space=pltpu.VMEM),
        scratch_shapes=[
            pltpu.VMEM((m, n), jnp.bfloat16),
            pltpu.VMEM((m, n), jnp.bfloat16),
            pltpu.SemaphoreType.DMA((nt,)),
            pltpu.SemaphoreType.DMA((nt,)),
        ],
        compiler_params=pltpu.CompilerParams(
            collective_id=0, vmem_limit_bytes=100 * 1024 * 1024
        ),
    )(n_send.reshape(1), wk, ws, x.astype(jnp.bfloat16), okw, osw)


# device time: 51120 ns/iter; 1.0693x vs baseline; 1.0693x over previous
import jax
import jax.numpy as jnp
from jax import lax
from jax.experimental import pallas as pl
from jax.experimental.pallas import tpu as pltpu

TILE = 256
WK = 768
WR = 512


def kernel(x, dest):
    m, n = x.shape
    nt = m // TILE

    my_y = lax.axis_index("y")

    is_send = (dest != my_y).astype(jnp.int32)
    cs = jnp.cumsum(is_send)
    n_send = cs[-1]
    nk = m - n_send
    idx = jnp.arange(m, dtype=jnp.int32)
    r_send = cs - is_send
    r_keep = idx - cs + is_send

    koff = jnp.where(my_y == 0, 0, n_send)
    okp = jnp.where(is_send == 0, koff + r_keep, -1).astype(jnp.int32)
    osr = jnp.where(is_send == 1, r_send, -1).astype(jnp.int32)

    def rank_windows(rank_vec, marked, target_vec, base):
        starts, wins = [], []
        total = jnp.sum(marked)
        for t in range(nt):
            rlo = jnp.clip(t * TILE - base, 0, jnp.maximum(total - 1, 0))
            match = (rank_vec == rlo) & (marked == 1)
            pos = jnp.sum(jnp.where(match, idx, 0))
            w_t = jnp.clip((pos // 128) * 128, 0, m - WK).astype(jnp.int32)
            starts.append(w_t)
            wins.append(lax.dynamic_slice(target_vec, (w_t,), (WK,)))
        return jnp.stack(starts), jnp.stack(wins)

    is_keep = 1 - is_send
    wk, okw = rank_windows(r_keep, is_keep, okp, koff)
    ws, osw = rank_windows(r_send, is_send, osr, 0)

    def body(ns_ref, wk_ref, ws_ref, x_ref, okw_ref, osw_ref, out_ref,
             xb_ref, xs_ref, sb_ref, send_sems, recv_sems):
        my_xx = lax.axis_index("x")
        my_yy = lax.axis_index("y")
        my_zz = lax.axis_index("z")
        peer = (my_xx, 1 - my_yy, my_zz)

        barrier = pltpu.get_barrier_semaphore()
        pl.semaphore_signal(
            barrier, inc=1, device_id=peer, device_id_type=pl.DeviceIdType.MESH
        )
        pl.semaphore_wait(barrier, 1)

        ns = ns_ref[0]
        nkk = m - ns
        koff_s = jnp.where(my_yy == 0, 0, ns)
        ro = jnp.where(my_yy == 0, nkk, 0)

        xb_ref[...] = x_ref[...].astype(jnp.bfloat16)

        def tile_rdma(t):
            return pltpu.make_async_remote_copy(
                src_ref=xs_ref.at[pl.ds(t * TILE, TILE)],
                dst_ref=sb_ref.at[pl.ds(t * TILE, TILE)],
                send_sem=send_sems.at[t],
                recv_sem=recv_sems.at[t],
                device_id=peer,
                device_id_type=pl.DeviceIdType.MESH,
            )

        def onehot_mm(win_ref, t, w_t, t0):
            d_iota = lax.broadcasted_iota(jnp.int32, (TILE, WK), 0) + t0
            p = (win_ref[t : t + 1, :] == d_iota).astype(jnp.bfloat16)
            return jax.lax.dot_general(
                p,
                xb_ref[pl.ds(w_t, WK), :],
                dimension_numbers=(((1,), (0,)), ((), ())),
                preferred_element_type=jnp.float32,
            )

        for t in range(nt):
            @pl.when(t * TILE < ns)
            def _():
                w_t = pl.multiple_of(ws_ref[t], 128)
                xs_ref[pl.ds(t * TILE, TILE), :] = onehot_mm(
                    osw_ref, t, w_t, t * TILE
                ).astype(jnp.bfloat16)
                tile_rdma(t).start()

        for t in range(nt):
            t0 = t * TILE
            has_keep = (t0 < koff_s + nkk) & (t0 + TILE > koff_s)

            @pl.when(has_keep)
            def _():
                w_t = pl.multiple_of(wk_ref[t], 128)
                out_ref[pl.ds(t0, TILE), :] = onehot_mm(okw_ref, t, w_t, t0)

        for t in range(nt):
            @pl.when(t * TILE < ns)
            def _():
                tile_rdma(t).wait_recv()

        for t in range(nt):
            t0 = t * TILE
            has_keep = (t0 < koff_s + nkk) & (t0 + TILE > koff_s)
            has_recv = (t0 < ro + ns) & (t0 + TILE > ro)
            wr_t = pl.multiple_of(
                jnp.clip((jnp.maximum(t0 - ro, 0) // 128) * 128, 0, m - WR),
                128,
            )
            d_iota = lax.broadcasted_iota(jnp.int32, (TILE, WR), 0) + t0
            j_iota = lax.broadcasted_iota(jnp.int32, (TILE, WR), 1) + wr_t
            band = (j_iota == d_iota - ro) & (j_iota < ns)

            def recv_mm():
                jrow = lax.broadcasted_iota(jnp.int32, (WR, 1), 0) + wr_t
                sbw = jnp.where(
                    jrow < ns, sb_ref[pl.ds(wr_t, WR), :], jnp.bfloat16(0)
                )
                return jax.lax.dot_general(
                    band.astype(jnp.bfloat16),
                    sbw,
                    dimension_numbers=(((1,), (0,)), ((), ())),
                    preferred_element_type=jnp.float32,
                )

            @pl.when(has_recv & jnp.logical_not(has_keep))
            def _():
                out_ref[pl.ds(t0, TILE), :] = recv_mm()

            @pl.when(has_recv & has_keep)
            def _():
                out_ref[pl.ds(t0, TILE), :] = (
                    out_ref[pl.ds(t0, TILE), :] + recv_mm()
                )

        for t in range(nt):
            @pl.when(t * TILE < ns)
            def _():
                tile_rdma(t).wait_send()

    return pl.pallas_call(
        body,
        out_shape=jax.ShapeDtypeStruct((m, n), x.dtype),
        in_specs=[
            pl.BlockSpec(memory_space=pltpu.SMEM),
            pl.BlockSpec(memory_space=pltpu.SMEM),
            pl.BlockSpec(memory_space=pltpu.SMEM),
            pl.BlockSpec(memory_space=pltpu.VMEM),
            pl.BlockSpec(memory_space=pltpu.VMEM),
            pl.BlockSpec(memory_space=pltpu.VMEM),
        ],
        out_specs=pl.BlockSpec(memory_space=pltpu.VMEM),
        scratch_shapes=[
            pltpu.VMEM((m, n), jnp.bfloat16),
            pltpu.VMEM((m, n), jnp.bfloat16),
            pltpu.VMEM((m, n), jnp.bfloat16),
            pltpu.SemaphoreType.DMA((nt,)),
            pltpu.SemaphoreType.DMA((nt,)),
        ],
        compiler_params=pltpu.CompilerParams(
            collective_id=0, vmem_limit_bytes=100 * 1024 * 1024
        ),
    )(n_send.reshape(1), wk, ws, x, okw, osw)


# device time: 50654 ns/iter; 1.0792x vs baseline; 1.0092x over previous
import jax
import jax.numpy as jnp
from jax import lax
from jax.experimental import pallas as pl
from jax.experimental.pallas import tpu as pltpu

TILE = 256
WK = 768
WR = 512


def kernel(x, dest):
    m, n = x.shape
    nt = m // TILE

    my_y = lax.axis_index("y")

    is_send = (dest != my_y).astype(jnp.int32)
    cs = jnp.cumsum(is_send)
    n_send = cs[-1]
    nk = m - n_send
    idx = jnp.arange(m, dtype=jnp.int32)
    r_send = cs - is_send
    r_keep = idx - cs + is_send

    koff = jnp.where(my_y == 0, 0, n_send)
    okp = jnp.where(is_send == 0, koff + r_keep, -1).astype(jnp.int32)
    osr = jnp.where(is_send == 1, r_send, -1).astype(jnp.int32)

    def rank_windows(rank_vec, marked, target_vec, base):
        starts, wins = [], []
        total = jnp.sum(marked)
        for t in range(nt):
            rlo = jnp.clip(t * TILE - base, 0, jnp.maximum(total - 1, 0))
            match = (rank_vec == rlo) & (marked == 1)
            pos = jnp.sum(jnp.where(match, idx, 0))
            w_t = jnp.clip((pos // 128) * 128, 0, m - WK).astype(jnp.int32)
            starts.append(w_t)
            wins.append(lax.dynamic_slice(target_vec, (w_t,), (WK,)))
        return jnp.stack(starts), jnp.stack(wins)

    is_keep = 1 - is_send
    wk, okw = rank_windows(r_keep, is_keep, okp, koff)
    ws, osw = rank_windows(r_send, is_send, osr, 0)

    def body(ns_ref, wk_ref, ws_ref, x_ref, okw_ref, osw_ref, out_ref,
             xb_ref, xs_ref, sb_ref, send_sems, recv_sems):
        my_xx = lax.axis_index("x")
        my_yy = lax.axis_index("y")
        my_zz = lax.axis_index("z")
        peer = (my_xx, 1 - my_yy, my_zz)

        barrier = pltpu.get_barrier_semaphore()
        pl.semaphore_signal(
            barrier, inc=1, device_id=peer, device_id_type=pl.DeviceIdType.MESH
        )
        pl.semaphore_wait(barrier, 1)

        ns = ns_ref[0]
        nkk = m - ns
        koff_s = jnp.where(my_yy == 0, 0, ns)
        ro = jnp.where(my_yy == 0, nkk, 0)

        xb_ref[...] = x_ref[...].astype(jnp.bfloat16)

        def tile_rdma(t):
            return pltpu.make_async_remote_copy(
                src_ref=xs_ref.at[pl.ds(t * TILE, TILE)],
                dst_ref=sb_ref.at[pl.ds(t * TILE, TILE)],
                send_sem=send_sems.at[t],
                recv_sem=recv_sems.at[t],
                device_id=peer,
                device_id_type=pl.DeviceIdType.MESH,
            )

        def onehot_mm(win_ref, t, w_t, t0):
            d_iota = lax.broadcasted_iota(jnp.int32, (TILE, WK), 0) + t0
            p = (win_ref[t : t + 1, :] == d_iota).astype(jnp.bfloat16)
            return jax.lax.dot_general(
                p,
                xb_ref[pl.ds(w_t, WK), :],
                dimension_numbers=(((1,), (0,)), ((), ())),
                preferred_element_type=jnp.float32,
            )

        for t in range(nt):
            @pl.when(t * TILE < ns)
            def _():
                w_t = pl.multiple_of(ws_ref[t], 128)
                xs_ref[pl.ds(t * TILE, TILE), :] = onehot_mm(
                    osw_ref, t, w_t, t * TILE
                ).astype(jnp.bfloat16)
                tile_rdma(t).start()

        for t in range(nt):
            t0 = t * TILE
            has_keep = (t0 < koff_s + nkk) & (t0 + TILE > koff_s)

            @pl.when(has_keep)
            def _():
                w_t = pl.multiple_of(wk_ref[t], 128)
                out_ref[pl.ds(t0, TILE), :] = onehot_mm(okw_ref, t, w_t, t0)

        for t in range(nt):
            t0 = t * TILE

            @pl.when(t0 < ns)
            def _():
                tile_rdma(t).wait_recv()
            has_keep = (t0 < koff_s + nkk) & (t0 + TILE > koff_s)
            has_recv = (t0 < ro + ns) & (t0 + TILE > ro)
            wr_t = pl.multiple_of(
                jnp.clip((jnp.maximum(t0 - ro, 0) // 128) * 128, 0, m - WR),
                128,
            )
            d_iota = lax.broadcasted_iota(jnp.int32, (TILE, WR), 0) + t0
            j_iota = lax.broadcasted_iota(jnp.int32, (TILE, WR), 1) + wr_t
            band = (j_iota == d_iota - ro) & (j_iota < ns)

            def recv_mm():
                jrow = lax.broadcasted_iota(jnp.int32, (WR, 1), 0) + wr_t
                bound = jnp.minimum(ns, t0 + TILE)
                sbw = jnp.where(
                    jrow < bound, sb_ref[pl.ds(wr_t, WR), :], jnp.bfloat16(0)
                )
                return jax.lax.dot_general(
                    band.astype(jnp.bfloat16),
                    sbw,
                    dimension_numbers=(((1,), (0,)), ((), ())),
                    preferred_element_type=jnp.float32,
                )

            @pl.when(has_recv & jnp.logical_not(has_keep))
            def _():
                out_ref[pl.ds(t0, TILE), :] = recv_mm()

            @pl.when(has_recv & has_keep)
            def _():
                out_ref[pl.ds(t0, TILE), :] = (
                    out_ref[pl.ds(t0, TILE), :] + recv_mm()
                )

        for t in range(nt):
            @pl.when(t * TILE < ns)
            def _():
                tile_rdma(t).wait_send()

    return pl.pallas_call(
        body,
        out_shape=jax.ShapeDtypeStruct((m, n), x.dtype),
        in_specs=[
            pl.BlockSpec(memory_space=pltpu.SMEM),
            pl.BlockSpec(memory_space=pltpu.SMEM),
            pl.BlockSpec(memory_space=pltpu.SMEM),
            pl.BlockSpec(memory_space=pltpu.VMEM),
            pl.BlockSpec(memory_space=pltpu.VMEM),
            pl.BlockSpec(memory_space=pltpu.VMEM),
        ],
        out_specs=pl.BlockSpec(memory_space=pltpu.VMEM),
        scratch_shapes=[
            pltpu.VMEM((m, n), jnp.bfloat16),
            pltpu.VMEM((m, n), jnp.bfloat16),
            pltpu.VMEM((m, n), jnp.bfloat16),
            pltpu.SemaphoreType.DMA((nt,)),
            pltpu.SemaphoreType.DMA((nt,)),
        ],
        compiler_params=pltpu.CompilerParams(
            collective_id=0, vmem_limit_bytes=100 * 1024 * 1024
        ),
    )(n_send.reshape(1), wk, ws, x, okw, osw)
